# Initial kernel scaffold; baseline (speedup 1.0000x reference)
import jax
import jax.numpy as jnp
from jax.experimental import pallas as pl


def kernel(adj1, adj2, gc1_W, gc1_b, gc2_W, gc2_b, gc3_W, gc3_b, gc4_W, gc4_b, l1_W, l1_b, l2_W, l2_b, l3_W, l3_b):
    raise NotImplementedError("write your pallas kernel here")


# speedup vs baseline: 1.2250x; 1.2250x over previous
import functools

import jax
import jax.numpy as jnp
from jax.experimental import pallas as pl
from jax.experimental.pallas import tpu as pltpu

N = 10000
NHID = 128

BM1 = 200
BM2 = 400
BMS = 2000


def _layer_body(adj_ref, y_ref, b_ref, wn_ref, x_ref, *rest,
                do_norm, emit_bf16, fuse_y):
    a = adj_ref[...]
    if emit_bf16:
        ab = a.astype(jnp.bfloat16)
        rest[0][...] = ab
        yn_ref = rest[1] if fuse_y else None
    else:
        ab = a
        yn_ref = rest[0] if fuse_y else None

    acc = jnp.dot(ab, y_ref[...], preferred_element_type=jnp.float32)
    h = jnp.maximum(acc + b_ref[...], 0.0)
    if do_norm:
        norm = jnp.sqrt(jnp.sum(h * h, axis=1, keepdims=True))
        h = h / jnp.maximum(norm, 1e-12)
    x_ref[...] = h
    if fuse_y:
        yn_ref[...] = jnp.dot(
            h.astype(jnp.bfloat16), wn_ref[...],
            preferred_element_type=jnp.float32).astype(jnp.bfloat16)


def _gcn_layer(adj, y, b, w_next, *, bm, do_norm, emit_bf16, fuse_y):
    nm = N // bm
    in_specs = [
        pl.BlockSpec((bm, N), lambda m: (m, 0)),
        pl.BlockSpec((N, NHID), lambda m: (0, 0)),
        pl.BlockSpec((1, NHID), lambda m: (0, 0)),
        pl.BlockSpec((NHID, NHID), lambda m: (0, 0)),
    ]
    out_shapes = [jax.ShapeDtypeStruct((N, NHID), jnp.float32)]
    out_specs = [pl.BlockSpec((bm, NHID), lambda m: (m, 0))]
    if emit_bf16:
        out_shapes.append(jax.ShapeDtypeStruct((N, N), jnp.bfloat16))
        out_specs.append(pl.BlockSpec((bm, N), lambda m: (m, 0)))
    if fuse_y:
        out_shapes.append(jax.ShapeDtypeStruct((N, NHID), jnp.bfloat16))
        out_specs.append(pl.BlockSpec((bm, NHID), lambda m: (m, 0)))

    body = functools.partial(_layer_body, do_norm=do_norm,
                             emit_bf16=emit_bf16, fuse_y=fuse_y)
    return pl.pallas_call(
        body,
        grid=(nm,),
        in_specs=in_specs,
        out_specs=out_specs,
        out_shape=out_shapes,
        compiler_params=pltpu.CompilerParams(
            dimension_semantics=("arbitrary",)),
    )(adj, y, b, w_next)


def _score_body(x1a, x2a, x3a, x4a, x1b, x2b, x3b, x4b,
                w1, b1, w2, b2, w3, b3, out_ref):
    def mlp(x_ref):
        h = jnp.dot(x_ref[...].astype(jnp.bfloat16), w1[...],
                    preferred_element_type=jnp.float32)
        h = jnp.maximum(h + b1[...], 0.0)
        h = jnp.dot(h.astype(jnp.bfloat16), w2[...],
                    preferred_element_type=jnp.float32)
        h = jnp.maximum(h + b2[...], 0.0)
        return jnp.dot(h.astype(jnp.bfloat16), w3[...],
                       preferred_element_type=jnp.float32) + b3[...]

    sa = mlp(x1a) + mlp(x2a) + mlp(x3a) + mlp(x4a)
    sb = mlp(x1b) + mlp(x2b) + mlp(x3b) + mlp(x4b)
    out_ref[...] = sa * sb


def _score(xs_a, xs_b, w1, b1, w2, b2, w3, b3, *, bm):
    nm = N // bm
    xspec = pl.BlockSpec((bm, NHID), lambda m: (m, 0))
    wspec = lambda shape: pl.BlockSpec(shape, lambda m: (0, 0))
    return pl.pallas_call(
        _score_body,
        grid=(nm,),
        in_specs=[xspec] * 8 + [
            wspec((NHID, 2 * NHID)), wspec((1, 2 * NHID)),
            wspec((2 * NHID, 2 * NHID)), wspec((1, 2 * NHID)),
            wspec((2 * NHID, 1)), wspec((1, 1)),
        ],
        out_specs=pl.BlockSpec((bm, 1), lambda m: (m, 0)),
        out_shape=jax.ShapeDtypeStruct((N, 1), jnp.float32),
        compiler_params=pltpu.CompilerParams(
            dimension_semantics=("arbitrary",)),
    )(*xs_a, *xs_b, w1, b1, w2, b2, w3, b3)


def _branch(adj, gc1_W, gc1_b, gc2_W, gc2_b, gc3_W, gc3_b, gc4_W, gc4_b):
    b1 = gc1_b.reshape(1, NHID)
    b2 = gc2_b.reshape(1, NHID)
    b3 = gc3_b.reshape(1, NHID)
    b4 = gc4_b.reshape(1, NHID)
    w2 = gc2_W.astype(jnp.bfloat16)
    w3 = gc3_W.astype(jnp.bfloat16)
    w4 = gc4_W.astype(jnp.bfloat16)
    w1 = gc1_W.astype(jnp.bfloat16)

    x1, adj_bf, y2 = _gcn_layer(adj, w1, b1, w2, bm=BM1,
                                do_norm=True, emit_bf16=True, fuse_y=True)
    x2, y3 = _gcn_layer(adj_bf, y2, b2, w3, bm=BM2,
                        do_norm=True, emit_bf16=False, fuse_y=True)
    x3, y4 = _gcn_layer(adj_bf, y3, b3, w4, bm=BM2,
                        do_norm=True, emit_bf16=False, fuse_y=True)
    (x4,) = _gcn_layer(adj_bf, y4, b4, w4, bm=BM2,
                       do_norm=False, emit_bf16=False, fuse_y=False)
    return x1, x2, x3, x4


def kernel(adj1, adj2, gc1_W, gc1_b, gc2_W, gc2_b, gc3_W, gc3_b,
           gc4_W, gc4_b, l1_W, l1_b, l2_W, l2_b, l3_W, l3_b):
    xs_a = _branch(adj1, gc1_W, gc1_b, gc2_W, gc2_b, gc3_W, gc3_b,
                   gc4_W, gc4_b)
    xs_b = _branch(adj2, gc1_W, gc1_b, gc2_W, gc2_b, gc3_W, gc3_b,
                   gc4_W, gc4_b)
    return _score(xs_a, xs_b,
                  l1_W.astype(jnp.bfloat16), l1_b.reshape(1, -1),
                  l2_W.astype(jnp.bfloat16), l2_b.reshape(1, -1),
                  l3_W.astype(jnp.bfloat16), l3_b.reshape(1, 1),
                  bm=BMS)


# speedup vs baseline: 1.2959x; 1.0579x over previous
import functools

import jax
import jax.numpy as jnp
from jax.experimental import pallas as pl
from jax.experimental.pallas import tpu as pltpu

N = 10000
NHID = 128

BM1 = 400
BM2 = 1000
BMS = 2000


def _layer_body(adj_ref, y_ref, b_ref, wn_ref, x_ref, *rest,
                do_norm, emit_bf16, fuse_y):
    a = adj_ref[...]
    if emit_bf16:
        ab = a.astype(jnp.bfloat16)
        rest[0][...] = ab
        yn_ref = rest[1] if fuse_y else None
    else:
        ab = a
        yn_ref = rest[0] if fuse_y else None

    acc = jnp.dot(ab, y_ref[...], preferred_element_type=jnp.float32)
    h = jnp.maximum(acc + b_ref[...], 0.0)
    if do_norm:
        norm = jnp.sqrt(jnp.sum(h * h, axis=1, keepdims=True))
        h = h / jnp.maximum(norm, 1e-12)
    x_ref[...] = h
    if fuse_y:
        yn_ref[...] = jnp.dot(
            h.astype(jnp.bfloat16), wn_ref[...],
            preferred_element_type=jnp.float32).astype(jnp.bfloat16)


def _gcn_layer(adj, y, b, w_next, *, bm, do_norm, emit_bf16, fuse_y):
    nm = N // bm
    in_specs = [
        pl.BlockSpec((bm, N), lambda m: (m, 0)),
        pl.BlockSpec((N, NHID), lambda m: (0, 0)),
        pl.BlockSpec((1, NHID), lambda m: (0, 0)),
        pl.BlockSpec((NHID, NHID), lambda m: (0, 0)),
    ]
    out_shapes = [jax.ShapeDtypeStruct((N, NHID), jnp.float32)]
    out_specs = [pl.BlockSpec((bm, NHID), lambda m: (m, 0))]
    if emit_bf16:
        out_shapes.append(jax.ShapeDtypeStruct((N, N), jnp.bfloat16))
        out_specs.append(pl.BlockSpec((bm, N), lambda m: (m, 0)))
    if fuse_y:
        out_shapes.append(jax.ShapeDtypeStruct((N, NHID), jnp.bfloat16))
        out_specs.append(pl.BlockSpec((bm, NHID), lambda m: (m, 0)))

    body = functools.partial(_layer_body, do_norm=do_norm,
                             emit_bf16=emit_bf16, fuse_y=fuse_y)
    return pl.pallas_call(
        body,
        grid=(nm,),
        in_specs=in_specs,
        out_specs=out_specs,
        out_shape=out_shapes,
        compiler_params=pltpu.CompilerParams(
            dimension_semantics=("arbitrary",)),
    )(adj, y, b, w_next)


def _score_body(x1a, x2a, x3a, x4a, x1b, x2b, x3b, x4b,
                w1, b1, w2, b2, w3, b3, out_ref):
    def mlp(x_ref):
        h = jnp.dot(x_ref[...].astype(jnp.bfloat16), w1[...],
                    preferred_element_type=jnp.float32)
        h = jnp.maximum(h + b1[...], 0.0)
        h = jnp.dot(h.astype(jnp.bfloat16), w2[...],
                    preferred_element_type=jnp.float32)
        h = jnp.maximum(h + b2[...], 0.0)
        return jnp.dot(h.astype(jnp.bfloat16), w3[...],
                       preferred_element_type=jnp.float32) + b3[...]

    sa = mlp(x1a) + mlp(x2a) + mlp(x3a) + mlp(x4a)
    sb = mlp(x1b) + mlp(x2b) + mlp(x3b) + mlp(x4b)
    out_ref[...] = sa * sb


def _score(xs_a, xs_b, w1, b1, w2, b2, w3, b3, *, bm):
    nm = N // bm
    xspec = pl.BlockSpec((bm, NHID), lambda m: (m, 0))
    wspec = lambda shape: pl.BlockSpec(shape, lambda m: (0, 0))
    return pl.pallas_call(
        _score_body,
        grid=(nm,),
        in_specs=[xspec] * 8 + [
            wspec((NHID, 2 * NHID)), wspec((1, 2 * NHID)),
            wspec((2 * NHID, 2 * NHID)), wspec((1, 2 * NHID)),
            wspec((2 * NHID, 1)), wspec((1, 1)),
        ],
        out_specs=pl.BlockSpec((bm, 1), lambda m: (m, 0)),
        out_shape=jax.ShapeDtypeStruct((N, 1), jnp.float32),
        compiler_params=pltpu.CompilerParams(
            dimension_semantics=("arbitrary",)),
    )(*xs_a, *xs_b, w1, b1, w2, b2, w3, b3)


def _branch(adj, gc1_W, gc1_b, gc2_W, gc2_b, gc3_W, gc3_b, gc4_W, gc4_b):
    b1 = gc1_b.reshape(1, NHID)
    b2 = gc2_b.reshape(1, NHID)
    b3 = gc3_b.reshape(1, NHID)
    b4 = gc4_b.reshape(1, NHID)
    w2 = gc2_W.astype(jnp.bfloat16)
    w3 = gc3_W.astype(jnp.bfloat16)
    w4 = gc4_W.astype(jnp.bfloat16)
    w1 = gc1_W.astype(jnp.bfloat16)

    x1, adj_bf, y2 = _gcn_layer(adj, w1, b1, w2, bm=BM1,
                                do_norm=True, emit_bf16=True, fuse_y=True)
    x2, y3 = _gcn_layer(adj_bf, y2, b2, w3, bm=BM2,
                        do_norm=True, emit_bf16=False, fuse_y=True)
    x3, y4 = _gcn_layer(adj_bf, y3, b3, w4, bm=BM2,
                        do_norm=True, emit_bf16=False, fuse_y=True)
    (x4,) = _gcn_layer(adj_bf, y4, b4, w4, bm=BM2,
                       do_norm=False, emit_bf16=False, fuse_y=False)
    return x1, x2, x3, x4


def kernel(adj1, adj2, gc1_W, gc1_b, gc2_W, gc2_b, gc3_W, gc3_b,
           gc4_W, gc4_b, l1_W, l1_b, l2_W, l2_b, l3_W, l3_b):
    xs_a = _branch(adj1, gc1_W, gc1_b, gc2_W, gc2_b, gc3_W, gc3_b,
                   gc4_W, gc4_b)
    xs_b = _branch(adj2, gc1_W, gc1_b, gc2_W, gc2_b, gc3_W, gc3_b,
                   gc4_W, gc4_b)
    return _score(xs_a, xs_b,
                  l1_W.astype(jnp.bfloat16), l1_b.reshape(1, -1),
                  l2_W.astype(jnp.bfloat16), l2_b.reshape(1, -1),
                  l3_W.astype(jnp.bfloat16), l3_b.reshape(1, 1),
                  bm=BMS)


# speedup vs baseline: 1.3243x; 1.0220x over previous
import functools

import jax
import jax.numpy as jnp
from jax.experimental import pallas as pl
from jax.experimental.pallas import tpu as pltpu

N = 10000
NHID = 128

BM1 = 400
BM2 = 1000


def _mlp(h, w1, b1, w2, b2, w3, b3):
    z = jnp.dot(h.astype(jnp.bfloat16), w1[...],
                preferred_element_type=jnp.float32)
    z = jnp.maximum(z + b1[...], 0.0)
    z = jnp.dot(z.astype(jnp.bfloat16), w2[...],
                preferred_element_type=jnp.float32)
    z = jnp.maximum(z + b2[...], 0.0)
    return jnp.dot(z.astype(jnp.bfloat16), w3[...],
                   preferred_element_type=jnp.float32) + b3[...]


def _l1_body(adj_ref, w1_ref, b1_ref, wn_ref,
             m1_ref, m1b_ref, m2_ref, m2b_ref, m3_ref, m3b_ref,
             adjb_ref, y2_ref, s1_ref):
    ab = adj_ref[...].astype(jnp.bfloat16)
    adjb_ref[...] = ab
    acc = jnp.dot(ab, w1_ref[...], preferred_element_type=jnp.float32)
    h = jnp.maximum(acc + b1_ref[...], 0.0)
    norm = jnp.sqrt(jnp.sum(h * h, axis=1, keepdims=True))
    h = h / jnp.maximum(norm, 1e-12)
    y2_ref[...] = jnp.dot(h.astype(jnp.bfloat16), wn_ref[...],
                          preferred_element_type=jnp.float32
                          ).astype(jnp.bfloat16)
    s1_ref[...] = _mlp(h, m1_ref, m1b_ref, m2_ref, m2b_ref, m3_ref, m3b_ref)


def _layer1(adj, w1, b1, w_next, mlp_ws):
    nm = N // BM1
    return pl.pallas_call(
        _l1_body,
        grid=(nm,),
        in_specs=[
            pl.BlockSpec((BM1, N), lambda m: (m, 0)),
            pl.BlockSpec((N, NHID), lambda m: (0, 0)),
            pl.BlockSpec((1, NHID), lambda m: (0, 0)),
            pl.BlockSpec((NHID, NHID), lambda m: (0, 0)),
            pl.BlockSpec((NHID, 2 * NHID), lambda m: (0, 0)),
            pl.BlockSpec((1, 2 * NHID), lambda m: (0, 0)),
            pl.BlockSpec((2 * NHID, 2 * NHID), lambda m: (0, 0)),
            pl.BlockSpec((1, 2 * NHID), lambda m: (0, 0)),
            pl.BlockSpec((2 * NHID, 1), lambda m: (0, 0)),
            pl.BlockSpec((1, 1), lambda m: (0, 0)),
        ],
        out_specs=[
            pl.BlockSpec((BM1, N), lambda m: (m, 0)),
            pl.BlockSpec((BM1, NHID), lambda m: (m, 0)),
            pl.BlockSpec((BM1, 1), lambda m: (m, 0)),
        ],
        out_shape=[
            jax.ShapeDtypeStruct((N, N), jnp.bfloat16),
            jax.ShapeDtypeStruct((N, NHID), jnp.bfloat16),
            jax.ShapeDtypeStruct((N, 1), jnp.float32),
        ],
        compiler_params=pltpu.CompilerParams(
            dimension_semantics=("arbitrary",)),
    )(adj, w1, b1, w_next, *mlp_ws)


def _l234_body(adjb_ref, y2_ref, s1_ref, bias_ref, ws_ref,
               m1_ref, m1b_ref, m2_ref, m2b_ref, m3_ref, m3b_ref,
               sother_ref, out_ref,
               ycur_ref, ynxt_ref, sacc_ref, *, with_other):
    l = pl.program_id(0)
    m = pl.program_id(1)

    @pl.when((l == 0) & (m == 0))
    def _():
        ycur_ref[...] = y2_ref[...]

    @pl.when((l > 0) & (m == 0))
    def _():
        ycur_ref[...] = ynxt_ref[...]

    acc = jnp.dot(adjb_ref[...], ycur_ref[...],
                  preferred_element_type=jnp.float32)
    h = jnp.maximum(acc + bias_ref[0], 0.0)
    norm = jnp.sqrt(jnp.sum(h * h, axis=1, keepdims=True))
    hn = h / jnp.maximum(norm, 1e-12)
    h = jnp.where(l <= 1, hn, h)

    rows = pl.ds(m * BM2, BM2)

    @pl.when(l <= 1)
    def _():
        ynxt_ref[rows, :] = jnp.dot(
            h.astype(jnp.bfloat16), ws_ref[0],
            preferred_element_type=jnp.float32).astype(jnp.bfloat16)

    z = _mlp(h, m1_ref, m1b_ref, m2_ref, m2b_ref, m3_ref, m3b_ref)

    @pl.when(l == 0)
    def _():
        sacc_ref[rows, :] = s1_ref[...] + z

    @pl.when(l == 1)
    def _():
        sacc_ref[rows, :] += z

    @pl.when(l == 2)
    def _():
        s = sacc_ref[rows, :] + z
        if with_other:
            s = s * sother_ref[...]
        out_ref[...] = s


def _layers234(adj_bf, y2, s1, biases, ws, mlp_ws, s_other):
    nm = N // BM2
    with_other = s_other is not None
    if s_other is None:
        s_other = s1
    body = functools.partial(_l234_body, with_other=with_other)
    row_spec = pl.BlockSpec((BM2, 1), lambda l, m: (m, 0))
    return pl.pallas_call(
        body,
        grid=(3, nm),
        in_specs=[
            pl.BlockSpec((BM2, N), lambda l, m: (m, 0)),
            pl.BlockSpec((N, NHID), lambda l, m: (0, 0)),
            row_spec,
            pl.BlockSpec((1, 1, NHID), lambda l, m: (l, 0, 0)),
            pl.BlockSpec((1, NHID, NHID),
                         lambda l, m: (jnp.minimum(l, 1), 0, 0)),
            pl.BlockSpec((NHID, 2 * NHID), lambda l, m: (0, 0)),
            pl.BlockSpec((1, 2 * NHID), lambda l, m: (0, 0)),
            pl.BlockSpec((2 * NHID, 2 * NHID), lambda l, m: (0, 0)),
            pl.BlockSpec((1, 2 * NHID), lambda l, m: (0, 0)),
            pl.BlockSpec((2 * NHID, 1), lambda l, m: (0, 0)),
            pl.BlockSpec((1, 1), lambda l, m: (0, 0)),
            row_spec,
        ],
        out_specs=pl.BlockSpec((BM2, 1), lambda l, m: (m, 0)),
        out_shape=jax.ShapeDtypeStruct((N, 1), jnp.float32),
        scratch_shapes=[
            pltpu.VMEM((N, NHID), jnp.bfloat16),
            pltpu.VMEM((N, NHID), jnp.bfloat16),
            pltpu.VMEM((N, 1), jnp.float32),
        ],
        compiler_params=pltpu.CompilerParams(
            dimension_semantics=("arbitrary", "arbitrary")),
    )(adj_bf, y2, s1, biases, ws, *mlp_ws, s_other)


def kernel(adj1, adj2, gc1_W, gc1_b, gc2_W, gc2_b, gc3_W, gc3_b,
           gc4_W, gc4_b, l1_W, l1_b, l2_W, l2_b, l3_W, l3_b):
    mlp_ws = (l1_W.astype(jnp.bfloat16), l1_b.reshape(1, -1),
              l2_W.astype(jnp.bfloat16), l2_b.reshape(1, -1),
              l3_W.astype(jnp.bfloat16), l3_b.reshape(1, 1))
    w1 = gc1_W.astype(jnp.bfloat16)
    w2 = gc2_W.astype(jnp.bfloat16)
    biases = jnp.stack([gc2_b, gc3_b, gc4_b]).reshape(3, 1, NHID)
    ws = jnp.stack([gc3_W.astype(jnp.bfloat16), gc4_W.astype(jnp.bfloat16)])
    b1 = gc1_b.reshape(1, NHID)

    adj1_bf, y2a, s1a = _layer1(adj1, w1, b1, w2, mlp_ws)
    s_a = _layers234(adj1_bf, y2a, s1a, biases, ws, mlp_ws, None)
    adj2_bf, y2b, s1b = _layer1(adj2, w1, b1, w2, mlp_ws)
    return _layers234(adj2_bf, y2b, s1b, biases, ws, mlp_ws, s_a)
